# initial kernel scaffold (unmeasured)
import jax
import jax.numpy as jnp
from jax import lax
from jax.experimental import pallas as pl
from jax.experimental.pallas import tpu as pltpu


def kernel(x, pi):
    def body(pi_ref, x_ref, out_ref, send_sem, recv_sem):
        my_x = lax.axis_index("x")
        my_y = lax.axis_index("y")
        my_z = lax.axis_index("z")
        tgt_x = pi_ref[my_x]

        barrier_sem = pltpu.get_barrier_semaphore()
        pl.semaphore_signal(
            barrier_sem,
            inc=1,
            device_id=(tgt_x, my_y, my_z),
            device_id_type=pl.DeviceIdType.MESH,
        )
        pl.semaphore_wait(barrier_sem, 1)

        rdma = pltpu.make_async_remote_copy(
            src_ref=x_ref,
            dst_ref=out_ref,
            send_sem=send_sem,
            recv_sem=recv_sem,
            device_id=(tgt_x, my_y, my_z),
            device_id_type=pl.DeviceIdType.MESH,
        )
        rdma.start()
        rdma.wait()

    return pl.pallas_call(
        body,
        out_shape=jax.ShapeDtypeStruct(x.shape, x.dtype),
        in_specs=[
            pl.BlockSpec(memory_space=pltpu.SMEM),
            pl.BlockSpec(memory_space=pltpu.ANY),
        ],
        out_specs=pl.BlockSpec(memory_space=pltpu.ANY),
        scratch_shapes=[
            pltpu.SemaphoreType.DMA,
            pltpu.SemaphoreType.DMA,
        ],
        compiler_params=pltpu.CompilerParams(collective_id=0),
    )(pi, x)


# baseline (device time: 101511 ns/iter reference)
import jax
import jax.numpy as jnp
from jax import lax
from jax.experimental import pallas as pl
from jax.experimental.pallas import tpu as pltpu


def kernel(x, pi):
    def body(pi_ref, x_ref, out_ref, send_sem, recv_sem):
        my_x = lax.axis_index("x")
        my_y = lax.axis_index("y")
        my_z = lax.axis_index("z")
        tgt_x = pi_ref[my_x]

        barrier_sem = pltpu.get_barrier_semaphore()
        pl.semaphore_signal(
            barrier_sem,
            inc=1,
            device_id=(tgt_x, my_y, my_z),
            device_id_type=pl.DeviceIdType.MESH,
        )
        pl.semaphore_wait(barrier_sem, 1)

        rdma = pltpu.make_async_remote_copy(
            src_ref=x_ref,
            dst_ref=out_ref,
            send_sem=send_sem,
            recv_sem=recv_sem,
            device_id=(tgt_x, my_y, my_z),
            device_id_type=pl.DeviceIdType.MESH,
        )
        rdma.start()
        rdma.wait()

    return pl.pallas_call(
        body,
        out_shape=jax.ShapeDtypeStruct(x.shape, x.dtype),
        in_specs=[
            pl.BlockSpec(memory_space=pltpu.SMEM),
            pl.BlockSpec(memory_space=pl.ANY),
        ],
        out_specs=pl.BlockSpec(memory_space=pl.ANY),
        scratch_shapes=[
            pltpu.SemaphoreType.DMA,
            pltpu.SemaphoreType.DMA,
        ],
        compiler_params=pltpu.CompilerParams(collective_id=0),
    )(pi, x)


# device time: 72535 ns/iter; 1.3995x vs baseline; 1.3995x over previous
import jax
import jax.numpy as jnp
from jax import lax
from jax.experimental import pallas as pl
from jax.experimental.pallas import tpu as pltpu

Q = 512
H = 256

SX, SYQ, SZQ, SYD, SZD = range(5)


def kernel(x, pi):
    def body(pi_ref, x_ref, out_ref, send_sems, recv_sems):
        my_x = lax.axis_index("x")
        my_y = lax.axis_index("y")
        my_z = lax.axis_index("z")
        tgt_x = pi_ref[my_x]

        y_nbr = (my_x, 1 - my_y, my_z)
        z_nbr = (my_x, my_y, 1 - my_z)

        def qrow(y, z):
            return (y * 2 + z) * Q

        barrier_sem = pltpu.get_barrier_semaphore()
        for dev in [(tgt_x, my_y, my_z), y_nbr, z_nbr]:
            pl.semaphore_signal(
                barrier_sem,
                inc=1,
                device_id=dev,
                device_id_type=pl.DeviceIdType.MESH,
            )
        pl.semaphore_wait(barrier_sem, 3)

        r_own = qrow(my_y, my_z)
        rdma_x = pltpu.make_async_remote_copy(
            src_ref=x_ref.at[:, pl.ds(r_own, Q), :],
            dst_ref=out_ref.at[:, pl.ds(r_own, Q), :],
            send_sem=send_sems.at[SX],
            recv_sem=recv_sems.at[SX],
            device_id=(tgt_x, my_y, my_z),
            device_id_type=pl.DeviceIdType.MESH,
        )
        rdma_x.start()
        rdma_x.wait_recv()

        rdma_yq = pltpu.make_async_remote_copy(
            src_ref=out_ref.at[:, pl.ds(r_own, Q), :],
            dst_ref=out_ref.at[:, pl.ds(r_own, Q), :],
            send_sem=send_sems.at[SYQ],
            recv_sem=recv_sems.at[SYQ],
            device_id=y_nbr,
            device_id_type=pl.DeviceIdType.MESH,
        )
        rdma_zq = pltpu.make_async_remote_copy(
            src_ref=out_ref.at[:, pl.ds(r_own, Q), :],
            dst_ref=out_ref.at[:, pl.ds(r_own, Q), :],
            send_sem=send_sems.at[SZQ],
            recv_sem=recv_sems.at[SZQ],
            device_id=z_nbr,
            device_id_type=pl.DeviceIdType.MESH,
        )
        rdma_yq.start()
        rdma_zq.start()
        rdma_yq.wait_recv()
        rdma_zq.wait_recv()

        r_zq = qrow(my_y, 1 - my_z)
        rdma_yd = pltpu.make_async_remote_copy(
            src_ref=out_ref.at[:, pl.ds(r_zq, H), :],
            dst_ref=out_ref.at[:, pl.ds(r_zq, H), :],
            send_sem=send_sems.at[SYD],
            recv_sem=recv_sems.at[SYD],
            device_id=y_nbr,
            device_id_type=pl.DeviceIdType.MESH,
        )
        r_yq = qrow(1 - my_y, my_z)
        rdma_zd = pltpu.make_async_remote_copy(
            src_ref=out_ref.at[:, pl.ds(r_yq + H, H), :],
            dst_ref=out_ref.at[:, pl.ds(r_yq + H, H), :],
            send_sem=send_sems.at[SZD],
            recv_sem=recv_sems.at[SZD],
            device_id=z_nbr,
            device_id_type=pl.DeviceIdType.MESH,
        )
        rdma_yd.start()
        rdma_zd.start()
        rdma_yd.wait_recv()
        rdma_zd.wait_recv()

        rdma_x.wait_send()
        rdma_yq.wait_send()
        rdma_zq.wait_send()
        rdma_yd.wait_send()
        rdma_zd.wait_send()

    return pl.pallas_call(
        body,
        out_shape=jax.ShapeDtypeStruct(x.shape, x.dtype),
        in_specs=[
            pl.BlockSpec(memory_space=pltpu.SMEM),
            pl.BlockSpec(memory_space=pl.ANY),
        ],
        out_specs=pl.BlockSpec(memory_space=pl.ANY),
        scratch_shapes=[
            pltpu.SemaphoreType.DMA((5,)),
            pltpu.SemaphoreType.DMA((5,)),
        ],
        compiler_params=pltpu.CompilerParams(collective_id=0),
    )(pi, x)


# device time: 53540 ns/iter; 1.8960x vs baseline; 1.3548x over previous
import jax
import jax.numpy as jnp
from jax import lax
from jax.experimental import pallas as pl
from jax.experimental.pallas import tpu as pltpu

Q = 512
C = 4
CH = Q // C

SX, SYQ, SZQ, SYD, SZD = range(5)


def kernel(x, pi):
    def body(pi_ref, x_ref, out_ref, send_sems, recv_sems):
        my_x = lax.axis_index("x")
        my_y = lax.axis_index("y")
        my_z = lax.axis_index("z")
        tgt_x = pi_ref[my_x]

        x_nbr = (tgt_x, my_y, my_z)
        y_nbr = (my_x, 1 - my_y, my_z)
        z_nbr = (my_x, my_y, 1 - my_z)

        def qrow(y, z):
            return (y * 2 + z) * Q

        r_own = qrow(my_y, my_z)
        r_yq = qrow(1 - my_y, my_z)
        r_zq = qrow(my_y, 1 - my_z)
        r_diag = qrow(1 - my_y, 1 - my_z)

        def copy(rows, nrows, dev, flow, c, src=None):
            src_ref = x_ref if src == "x" else out_ref
            return pltpu.make_async_remote_copy(
                src_ref=src_ref.at[:, pl.ds(rows, nrows), :],
                dst_ref=out_ref.at[:, pl.ds(rows, nrows), :],
                send_sem=send_sems.at[flow, c],
                recv_sem=recv_sems.at[flow, c],
                device_id=dev,
                device_id_type=pl.DeviceIdType.MESH,
            )

        barrier_sem = pltpu.get_barrier_semaphore()
        for dev in [x_nbr, y_nbr, z_nbr]:
            pl.semaphore_signal(
                barrier_sem,
                inc=1,
                device_id=dev,
                device_id_type=pl.DeviceIdType.MESH,
            )
        pl.semaphore_wait(barrier_sem, 3)

        sx = []
        for c in range(C):
            d = copy(r_own + c * CH, CH, x_nbr, SX, c, src="x")
            d.start()
            sx.append(d)

        syq, szq, syd, szd = [], [], [], []
        ryq = [copy(r_yq + c * CH, CH, y_nbr, SYQ, c) for c in range(C)]
        rzq = [copy(r_zq + c * CH, CH, z_nbr, SZQ, c) for c in range(C)]
        ryd = [copy(r_diag + c * CH, CH, y_nbr, SYD, c) for c in range(C // 2)]
        rzd = [copy(r_diag + c * CH, CH, z_nbr, SZD, c) for c in range(C // 2, C)]

        for c in range(C):
            sx[c].wait_recv()
            dy = copy(r_own + c * CH, CH, y_nbr, SYQ, c)
            dz = copy(r_own + c * CH, CH, z_nbr, SZQ, c)
            dy.start()
            dz.start()
            syq.append(dy)
            szq.append(dz)

        for c in range(C):
            rzq[c].wait_recv()
            if c < C // 2:
                d = copy(r_zq + c * CH, CH, y_nbr, SYD, c)
                d.start()
                syd.append(d)
            ryq[c].wait_recv()
            if c >= C // 2:
                d = copy(r_yq + c * CH, CH, z_nbr, SZD, c)
                d.start()
                szd.append(d)

        for d in ryd:
            d.wait_recv()
        for d in rzd:
            d.wait_recv()

        for d in sx + syq + szq + syd + szd:
            d.wait_send()

    return pl.pallas_call(
        body,
        out_shape=jax.ShapeDtypeStruct(x.shape, x.dtype),
        in_specs=[
            pl.BlockSpec(memory_space=pltpu.SMEM),
            pl.BlockSpec(memory_space=pl.ANY),
        ],
        out_specs=pl.BlockSpec(memory_space=pl.ANY),
        scratch_shapes=[
            pltpu.SemaphoreType.DMA((5, C)),
            pltpu.SemaphoreType.DMA((5, C)),
        ],
        compiler_params=pltpu.CompilerParams(collective_id=0),
    )(pi, x)


# device time: 52123 ns/iter; 1.9475x vs baseline; 1.0272x over previous
import jax
import jax.numpy as jnp
from jax import lax
from jax.experimental import pallas as pl
from jax.experimental.pallas import tpu as pltpu

Q = 512
C = 4
CH = Q // C
XD = 176
FD = 168

SX, SXE, SYQ, SZQ, SYD, SZD = range(6)


def kernel(x, pi):
    def body(pi_ref, x_ref, out_ref, send_sems, recv_sems):
        my_x = lax.axis_index("x")
        my_y = lax.axis_index("y")
        my_z = lax.axis_index("z")
        tgt_x = pi_ref[my_x]

        x_nbr = (tgt_x, my_y, my_z)
        y_nbr = (my_x, 1 - my_y, my_z)
        z_nbr = (my_x, my_y, 1 - my_z)

        def qrow(y, z):
            return (y * 2 + z) * Q

        r_own = qrow(my_y, my_z)
        r_yq = qrow(1 - my_y, my_z)
        r_zq = qrow(my_y, 1 - my_z)
        r_diag = qrow(1 - my_y, 1 - my_z)

        def copy(rows, nrows, dev, flow, c, src=None):
            src_ref = x_ref if src == "x" else out_ref
            return pltpu.make_async_remote_copy(
                src_ref=src_ref.at[:, pl.ds(rows, nrows), :],
                dst_ref=out_ref.at[:, pl.ds(rows, nrows), :],
                send_sem=send_sems.at[flow, c],
                recv_sem=recv_sems.at[flow, c],
                device_id=dev,
                device_id_type=pl.DeviceIdType.MESH,
            )

        barrier_sem = pltpu.get_barrier_semaphore()
        for dev in [x_nbr, y_nbr, z_nbr]:
            pl.semaphore_signal(
                barrier_sem,
                inc=1,
                device_id=dev,
                device_id_type=pl.DeviceIdType.MESH,
            )
        pl.semaphore_wait(barrier_sem, 3)

        sx = []
        for c in range(C):
            d = copy(r_own + c * CH, CH, x_nbr, SX, c, src="x")
            d.start()
            sx.append(d)
        sxe = copy(r_diag, XD, x_nbr, SXE, 0, src="x")
        sxe.start()

        ryq = [copy(r_yq + c * CH, CH, y_nbr, SYQ, c) for c in range(C)]
        rzq = [copy(r_zq + c * CH, CH, z_nbr, SZQ, c) for c in range(C)]
        rxe = copy(r_diag, XD, x_nbr, SXE, 0)
        ryd = copy(r_diag + XD, FD, y_nbr, SYD, 0)
        rzd = copy(r_diag + XD + FD, FD, z_nbr, SZD, 0)

        syq, szq = [], []
        for c in range(C):
            sx[c].wait_recv()
            dy = copy(r_own + c * CH, CH, y_nbr, SYQ, c)
            dz = copy(r_own + c * CH, CH, z_nbr, SZQ, c)
            dy.start()
            dz.start()
            syq.append(dy)
            szq.append(dz)

        syd = szd = None
        for c in range(C):
            rzq[c].wait_recv()
            if c == 2:
                syd = copy(r_zq + XD, FD, y_nbr, SYD, 0)
                syd.start()
            ryq[c].wait_recv()
            if c == 3:
                szd = copy(r_yq + XD + FD, FD, z_nbr, SZD, 0)
                szd.start()

        rxe.wait_recv()
        ryd.wait_recv()
        rzd.wait_recv()

        for d in sx + syq + szq + [sxe, syd, szd]:
            d.wait_send()

    return pl.pallas_call(
        body,
        out_shape=jax.ShapeDtypeStruct(x.shape, x.dtype),
        in_specs=[
            pl.BlockSpec(memory_space=pltpu.SMEM),
            pl.BlockSpec(memory_space=pl.ANY),
        ],
        out_specs=pl.BlockSpec(memory_space=pl.ANY),
        scratch_shapes=[
            pltpu.SemaphoreType.DMA((6, C)),
            pltpu.SemaphoreType.DMA((6, C)),
        ],
        compiler_params=pltpu.CompilerParams(collective_id=0),
    )(pi, x)


# device time: 49832 ns/iter; 2.0371x vs baseline; 1.0460x over previous
import jax
import jax.numpy as jnp
from jax import lax
from jax.experimental import pallas as pl
from jax.experimental.pallas import tpu as pltpu

Q = 512
C = 8
CH = Q // C
XD = 176
FD = 168

SX, SXE, SYQ, SZQ, SYD, SZD = range(6)


def kernel(x, pi):
    def body(pi_ref, x_ref, out_ref, send_sems, recv_sems):
        my_x = lax.axis_index("x")
        my_y = lax.axis_index("y")
        my_z = lax.axis_index("z")
        tgt_x = pi_ref[my_x]

        x_nbr = (tgt_x, my_y, my_z)
        y_nbr = (my_x, 1 - my_y, my_z)
        z_nbr = (my_x, my_y, 1 - my_z)

        def qrow(y, z):
            return (y * 2 + z) * Q

        r_own = qrow(my_y, my_z)
        r_yq = qrow(1 - my_y, my_z)
        r_zq = qrow(my_y, 1 - my_z)
        r_diag = qrow(1 - my_y, 1 - my_z)

        def copy(rows, nrows, dev, flow, c, src=None):
            src_ref = x_ref if src == "x" else out_ref
            return pltpu.make_async_remote_copy(
                src_ref=src_ref.at[:, pl.ds(rows, nrows), :],
                dst_ref=out_ref.at[:, pl.ds(rows, nrows), :],
                send_sem=send_sems.at[flow, c],
                recv_sem=recv_sems.at[flow, c],
                device_id=dev,
                device_id_type=pl.DeviceIdType.MESH,
            )

        barrier_sem = pltpu.get_barrier_semaphore()
        for dev in [x_nbr, y_nbr, z_nbr]:
            pl.semaphore_signal(
                barrier_sem,
                inc=1,
                device_id=dev,
                device_id_type=pl.DeviceIdType.MESH,
            )
        pl.semaphore_wait(barrier_sem, 3)

        sx = []
        for c in range(C):
            d = copy(r_own + c * CH, CH, x_nbr, SX, c, src="x")
            d.start()
            sx.append(d)
        sxe = copy(r_diag, XD, x_nbr, SXE, 0, src="x")
        sxe.start()

        ryq = [copy(r_yq + c * CH, CH, y_nbr, SYQ, c) for c in range(C)]
        rzq = [copy(r_zq + c * CH, CH, z_nbr, SZQ, c) for c in range(C)]
        rxe = copy(r_diag, XD, x_nbr, SXE, 0)
        ryd = copy(r_diag + XD, FD, y_nbr, SYD, 0)
        rzd = copy(r_diag + XD + FD, FD, z_nbr, SZD, 0)

        syq, szq = [], []
        for c in range(C):
            sx[c].wait_recv()
            dy = copy(r_own + c * CH, CH, y_nbr, SYQ, c)
            dz = copy(r_own + c * CH, CH, z_nbr, SZQ, c)
            dy.start()
            dz.start()
            syq.append(dy)
            szq.append(dz)

        yd_after = (XD + FD - 1) // CH
        syd = szd = None
        for c in range(C):
            rzq[c].wait_recv()
            if c == yd_after:
                syd = copy(r_zq + XD, FD, y_nbr, SYD, 0)
                syd.start()
            ryq[c].wait_recv()
            if c == C - 1:
                szd = copy(r_yq + XD + FD, FD, z_nbr, SZD, 0)
                szd.start()

        rxe.wait_recv()
        ryd.wait_recv()
        rzd.wait_recv()

        for d in sx + syq + szq + [sxe, syd, szd]:
            d.wait_send()

    return pl.pallas_call(
        body,
        out_shape=jax.ShapeDtypeStruct(x.shape, x.dtype),
        in_specs=[
            pl.BlockSpec(memory_space=pltpu.SMEM),
            pl.BlockSpec(memory_space=pl.ANY),
        ],
        out_specs=pl.BlockSpec(memory_space=pl.ANY),
        scratch_shapes=[
            pltpu.SemaphoreType.DMA((6, C)),
            pltpu.SemaphoreType.DMA((6, C)),
        ],
        compiler_params=pltpu.CompilerParams(collective_id=0),
    )(pi, x)


# device time: 49312 ns/iter; 2.0585x vs baseline; 1.0105x over previous
import jax
import jax.numpy as jnp
from jax import lax
from jax.experimental import pallas as pl
from jax.experimental.pallas import tpu as pltpu

Q = 512
C = 16
CH = Q // C
XD = 176
FD = 168

SX, SXE, SYQ, SZQ, SYD, SZD = range(6)


def kernel(x, pi):
    def body(pi_ref, x_ref, out_ref, send_sems, recv_sems):
        my_x = lax.axis_index("x")
        my_y = lax.axis_index("y")
        my_z = lax.axis_index("z")
        tgt_x = pi_ref[my_x]

        x_nbr = (tgt_x, my_y, my_z)
        y_nbr = (my_x, 1 - my_y, my_z)
        z_nbr = (my_x, my_y, 1 - my_z)

        def qrow(y, z):
            return (y * 2 + z) * Q

        r_own = qrow(my_y, my_z)
        r_yq = qrow(1 - my_y, my_z)
        r_zq = qrow(my_y, 1 - my_z)
        r_diag = qrow(1 - my_y, 1 - my_z)

        def copy(rows, nrows, dev, flow, c, src=None):
            src_ref = x_ref if src == "x" else out_ref
            return pltpu.make_async_remote_copy(
                src_ref=src_ref.at[:, pl.ds(rows, nrows), :],
                dst_ref=out_ref.at[:, pl.ds(rows, nrows), :],
                send_sem=send_sems.at[flow, c],
                recv_sem=recv_sems.at[flow, c],
                device_id=dev,
                device_id_type=pl.DeviceIdType.MESH,
            )

        barrier_sem = pltpu.get_barrier_semaphore()
        for dev in [x_nbr, y_nbr, z_nbr]:
            pl.semaphore_signal(
                barrier_sem,
                inc=1,
                device_id=dev,
                device_id_type=pl.DeviceIdType.MESH,
            )
        pl.semaphore_wait(barrier_sem, 3)

        sx = []
        for c in range(C):
            d = copy(r_own + c * CH, CH, x_nbr, SX, c, src="x")
            d.start()
            sx.append(d)
        sxe = copy(r_diag, XD, x_nbr, SXE, 0, src="x")
        sxe.start()

        ryq = [copy(r_yq + c * CH, CH, y_nbr, SYQ, c) for c in range(C)]
        rzq = [copy(r_zq + c * CH, CH, z_nbr, SZQ, c) for c in range(C)]
        rxe = copy(r_diag, XD, x_nbr, SXE, 0)
        ryd = copy(r_diag + XD, FD, y_nbr, SYD, 0)
        rzd = copy(r_diag + XD + FD, FD, z_nbr, SZD, 0)

        syq, szq = [], []
        for c in range(C):
            sx[c].wait_recv()
            dy = copy(r_own + c * CH, CH, y_nbr, SYQ, c)
            dz = copy(r_own + c * CH, CH, z_nbr, SZQ, c)
            dy.start()
            dz.start()
            syq.append(dy)
            szq.append(dz)

        yd_after = (XD + FD - 1) // CH
        syd = szd = None
        for c in range(C):
            rzq[c].wait_recv()
            if c == yd_after:
                syd = copy(r_zq + XD, FD, y_nbr, SYD, 0)
                syd.start()
            ryq[c].wait_recv()
            if c == C - 1:
                szd = copy(r_yq + XD + FD, FD, z_nbr, SZD, 0)
                szd.start()

        rxe.wait_recv()
        ryd.wait_recv()
        rzd.wait_recv()

        for d in sx + syq + szq + [sxe, syd, szd]:
            d.wait_send()

    return pl.pallas_call(
        body,
        out_shape=jax.ShapeDtypeStruct(x.shape, x.dtype),
        in_specs=[
            pl.BlockSpec(memory_space=pltpu.SMEM),
            pl.BlockSpec(memory_space=pl.ANY),
        ],
        out_specs=pl.BlockSpec(memory_space=pl.ANY),
        scratch_shapes=[
            pltpu.SemaphoreType.DMA((6, C)),
            pltpu.SemaphoreType.DMA((6, C)),
        ],
        compiler_params=pltpu.CompilerParams(collective_id=0),
    )(pi, x)
